# baseline (speedup 1.0000x reference)
import jax
import jax.numpy as jnp
from jax import lax
from jax.experimental import pallas as pl
from jax.experimental.pallas import tpu as pltpu
from jax.experimental.pallas import tpu_sc as plsc

_B, _S, _D = 16384, 50, 64
_V = 1000000
_NC, _NS = 2, 16
_NW = _NC * _NS
_BW = _B // _NW
_BLK = 128
_NBLK = _BW // _BLK
_NCH = _S * _NBLK
_L = 16


def _body(t2, xt, out, idxall, idx2, gbuf, tbuf, sem_g, sem_o):
    wid = lax.axis_index("s") * _NC + lax.axis_index("c")
    wb0 = wid * _BW

    pltpu.sync_copy(xt.at[:, pl.ds(wb0, _BW)], idxall)

    iota = lax.iota(jnp.int32, _L)

    def sblk(i):
        return lax.div(i, _NBLK), lax.rem(i, _NBLK)

    def gather_desc(b):
        return pltpu.make_async_copy(t2.at[idx2.at[b]], gbuf.at[b], sem_g.at[b])

    def prep_and_start(i, b):
        s, blk = sblk(i)
        col0 = blk * _BLK
        for k in range(_BLK // _L):
            v = idxall[s, pl.ds(col0 + k * _L, _L)]
            idx2[b, pl.ds(k * _L, _L)] = lax.shift_right_logical(v, 1)
        gather_desc(b).start()

    def write_desc(i, b):
        s, blk = sblk(i)
        return pltpu.make_async_copy(
            tbuf.at[b], out.at[s, :, pl.ds(wb0 + blk * _BLK, _BLK)], sem_o.at[b]
        )

    def transpose_chunk(i, b):
        s, blk = sblk(i)
        col0 = blk * _BLK
        for k in range(_BLK // _L):
            idx16 = idxall[s, pl.ds(col0 + k * _L, _L)]
            h64 = lax.shift_left(lax.bitwise_and(idx16, 1), 6)
            rows = iota + (k * _L)
            for d in range(_D):
                vals = plsc.load_gather(gbuf.at[b], [rows, h64 + d])
                tbuf[b, d, pl.ds(k * _L, _L)] = vals

    prep_and_start(0, 0)
    prep_and_start(1, 1)

    def half(i, b):
        gather_desc(b).wait()

        @pl.when(i >= 2)
        def _():
            write_desc(i - 2, b).wait()

        transpose_chunk(i, b)
        write_desc(i, b).start()

        @pl.when(i + 2 < _NCH)
        def _():
            prep_and_start(i + 2, b)

    def step(j, _):
        half(2 * j, 0)
        half(2 * j + 1, 1)
        return 0

    lax.fori_loop(0, _NCH // 2, step, 0)
    write_desc(_NCH - 2, 0).wait()
    write_desc(_NCH - 1, 1).wait()


@jax.jit
def _embed(xt, t2):
    mesh = plsc.VectorSubcoreMesh(
        core_axis_name="c", subcore_axis_name="s",
        num_cores=_NC, num_subcores=_NS,
    )
    fn = pl.kernel(
        _body,
        out_type=jax.ShapeDtypeStruct((_S, _D, _B), jnp.float32),
        mesh=mesh,
        scratch_types=[
            pltpu.VMEM((_S, _BW), jnp.int32),
            pltpu.VMEM((2, _BLK), jnp.int32),
            pltpu.VMEM((2, _BLK, 2 * _D), jnp.float32),
            pltpu.VMEM((2, _D, _BLK), jnp.float32),
            pltpu.SemaphoreType.DMA((2,)),
            pltpu.SemaphoreType.DMA((2,)),
        ],
        compiler_params=pltpu.CompilerParams(
            use_tc_tiling_on_sc=True, needs_layout_passes=False
        ),
    )
    return fn(t2, xt)


def kernel(x, embedding_matrix):
    xt = jnp.transpose(x).astype(jnp.int32)
    t2 = embedding_matrix.reshape(_V // 2, 2 * _D)
    o = _embed(xt, t2)
    return jnp.transpose(o, (2, 0, 1))


# speedup vs baseline: 1.2065x; 1.2065x over previous
import jax
import jax.numpy as jnp
from jax import lax
from jax.experimental import pallas as pl
from jax.experimental.pallas import tpu as pltpu
from jax.experimental.pallas import tpu_sc as plsc

_B, _S, _D = 16384, 50, 64
_V = 1000000
_NC, _NS = 2, 16
_NW = _NC * _NS
_BW = _B // _NW
_BLK = 128
_NBLK = _BW // _BLK
_NCH = _S * _NBLK
_L = 16


def _body(t2, xt, out, idxall, idx2a, idx2b, gbufa, gbufb, tbufa, tbufb, sem_g, sem_o):
    idx2 = (idx2a, idx2b)
    gbuf = (gbufa, gbufb)
    tbuf = (tbufa, tbufb)
    wid = lax.axis_index("s") * _NC + lax.axis_index("c")
    wb0 = wid * _BW

    pltpu.sync_copy(xt.at[:, pl.ds(wb0, _BW)], idxall)

    iota = lax.iota(jnp.int32, _L)

    def sblk(i):
        return lax.div(i, _NBLK), lax.rem(i, _NBLK)

    def gather_desc(b):
        return pltpu.make_async_copy(t2.at[idx2[b]], gbuf[b], sem_g.at[b])

    def prep_and_start(i, b):
        s, blk = sblk(i)
        col0 = blk * _BLK
        for k in range(_BLK // _L):
            v = idxall[s, pl.ds(col0 + k * _L, _L)]
            idx2[b][pl.ds(k * _L, _L)] = lax.shift_right_logical(v, 1)
        gather_desc(b).start()

    def write_desc(i, b):
        s, blk = sblk(i)
        return pltpu.make_async_copy(
            tbuf[b].at[:], out.at[s, :, pl.ds(wb0 + blk * _BLK, _BLK)], sem_o.at[b]
        )

    drows = [iota + k * _L for k in range(_D // _L)]

    def transpose_chunk(i, b):
        s, blk = sblk(i)
        col0 = blk * _BLK

        def block(jb, _):
            v16 = idxall[s, pl.ds(col0 + jb * _L, _L)]
            h16 = lax.shift_left(lax.bitwise_and(v16, 1), 6)
            for jj in range(_L):
                j = jb * _L + jj
                h = h16[jj]
                cols = lax.broadcast(j, (_L,))
                for k in range(_D // _L):
                    v = gbuf[b][j, pl.ds(h + k * _L, _L)]
                    plsc.store_scatter(tbuf[b], [drows[k], cols], v)
            return 0

        lax.fori_loop(0, _BLK // _L, block, 0)

    prep_and_start(0, 0)
    prep_and_start(1, 1)

    def half(i, b):
        gather_desc(b).wait()

        @pl.when(i >= 2)
        def _():
            write_desc(i - 2, b).wait()

        transpose_chunk(i, b)
        write_desc(i, b).start()

        @pl.when(i + 2 < _NCH)
        def _():
            prep_and_start(i + 2, b)

    def step(j, _):
        half(2 * j, 0)
        half(2 * j + 1, 1)
        return 0

    lax.fori_loop(0, _NCH // 2, step, 0)
    write_desc(_NCH - 2, 0).wait()
    write_desc(_NCH - 1, 1).wait()


@jax.jit
def _embed(xt, t2):
    mesh = plsc.VectorSubcoreMesh(
        core_axis_name="c", subcore_axis_name="s",
        num_cores=_NC, num_subcores=_NS,
    )
    fn = pl.kernel(
        _body,
        out_type=jax.ShapeDtypeStruct((_S, _D, _B), jnp.float32),
        mesh=mesh,
        scratch_types=[
            pltpu.VMEM((_S, _BW), jnp.int32),
            pltpu.VMEM((_BLK,), jnp.int32),
            pltpu.VMEM((_BLK,), jnp.int32),
            pltpu.VMEM((_BLK, 2 * _D), jnp.float32),
            pltpu.VMEM((_BLK, 2 * _D), jnp.float32),
            pltpu.VMEM((_D, _BLK), jnp.float32),
            pltpu.VMEM((_D, _BLK), jnp.float32),
            pltpu.SemaphoreType.DMA((2,)),
            pltpu.SemaphoreType.DMA((2,)),
        ],
        compiler_params=pltpu.CompilerParams(
            use_tc_tiling_on_sc=True, needs_layout_passes=False
        ),
    )
    return fn(t2, xt)


def kernel(x, embedding_matrix):
    xt = jnp.transpose(x).astype(jnp.int32)
    t2 = embedding_matrix.reshape(_V // 2, 2 * _D)
    o = _embed(xt, t2)
    return jnp.transpose(o, (2, 0, 1))


# speedup vs baseline: 1.5595x; 1.2926x over previous
import jax
import jax.numpy as jnp
from jax import lax
from jax.experimental import pallas as pl
from jax.experimental.pallas import tpu as pltpu
from jax.experimental.pallas import tpu_sc as plsc

_B, _S = 16384, 50
_D = 64
_TOTAL = _B * _S
_NC, _NS = 2, 16
_NW = _NC * _NS
_PER_W = _TOTAL // _NW
_BPC = 8
_CHUNK = _BPC * _S
_N_CHUNKS = _PER_W // _CHUNK
_B_PER_W = _B // _NW


def _body(table_hbm, idx_hbm, out_hbm, idx_v, rows_v, sem_g, sem_o):
    wid = lax.axis_index("s") * _NC + lax.axis_index("c")
    base = wid * _PER_W

    pltpu.sync_copy(idx_hbm.at[pl.ds(base, _PER_W)], idx_v)

    def gather_desc(i, b):
        off = pl.multiple_of(i * _CHUNK, _CHUNK)
        return pltpu.make_async_copy(
            table_hbm.at[idx_v.at[pl.ds(off, _CHUNK)]], rows_v.at[b], sem_g.at[b]
        )

    def write_start(i, b):
        wb0 = wid * _B_PER_W
        for k in range(_BPC):
            pltpu.make_async_copy(
                rows_v.at[b, pl.ds(k * _S, _S)],
                out_hbm.at[wb0 + i * _BPC + k],
                sem_o.at[b],
            ).start()

    def write_wait(i, b):
        for k in range(_BPC):
            pltpu.make_async_copy(
                rows_v.at[b, pl.ds(k * _S, _S)],
                out_hbm.at[wid * _B_PER_W + i * _BPC + k],
                sem_o.at[b],
            ).wait()

    gather_desc(0, 0).start()
    gather_desc(1, 1).start()

    def half_step(i, b):
        gather_desc(i, b).wait()
        write_start(i, b)

        @pl.when(i + 2 < _N_CHUNKS)
        def _():
            write_wait(i, b)
            gather_desc(i + 2, b).start()

    def step(j, _):
        half_step(2 * j, 0)
        half_step(2 * j + 1, 1)
        return 0

    lax.fori_loop(0, _N_CHUNKS // 2, step, 0)
    write_wait(_N_CHUNKS - 2, 0)
    write_wait(_N_CHUNKS - 1, 1)


@jax.jit
def _embed(x_flat, table):
    mesh = plsc.VectorSubcoreMesh(
        core_axis_name="c", subcore_axis_name="s",
        num_cores=_NC, num_subcores=_NS,
    )
    fn = pl.kernel(
        _body,
        out_type=jax.ShapeDtypeStruct((_B, _S, _D), jnp.float32),
        mesh=mesh,
        scratch_types=[
            pltpu.VMEM((_PER_W,), jnp.int32),
            pltpu.VMEM((2, _CHUNK, _D), jnp.float32),
            pltpu.SemaphoreType.DMA((2,)),
            pltpu.SemaphoreType.DMA((2,)),
        ],
        compiler_params=pltpu.CompilerParams(use_tc_tiling_on_sc=False),
    )
    return fn(table, x_flat)


def kernel(x, embedding_matrix):
    x_flat = x.reshape(-1).astype(jnp.int32)
    return _embed(x_flat, embedding_matrix)
